# Initial kernel scaffold; baseline (speedup 1.0000x reference)
import jax
import jax.numpy as jnp
from jax.experimental import pallas as pl


def kernel(pts):
    raise NotImplementedError("write your pallas kernel here")


# speedup vs baseline: 1.9350x; 1.9350x over previous
import jax
import jax.numpy as jnp
from jax.experimental import pallas as pl
from jax.experimental.pallas import tpu as pltpu

B, N, G, K = 8, 8192, 256, 32


def _fps_body(xt_ref, cent_ref):
    x = xt_ref[0]
    y = xt_ref[1]
    z = xt_ref[2]
    col = jax.lax.broadcasted_iota(jnp.int32, (B, N), 1)
    col_g = jax.lax.broadcasted_iota(jnp.int32, (B, G), 1)

    def step(i, carry):
        dist_acc, far, cxa, cya, cza = carry
        m = col == far
        cx = jnp.sum(jnp.where(m, x, 0.0), axis=1, keepdims=True)
        cy = jnp.sum(jnp.where(m, y, 0.0), axis=1, keepdims=True)
        cz = jnp.sum(jnp.where(m, z, 0.0), axis=1, keepdims=True)
        sel = col_g == i
        cxa = jnp.where(sel, cx, cxa)
        cya = jnp.where(sel, cy, cya)
        cza = jnp.where(sel, cz, cza)
        dx = x - cx
        dy = y - cy
        dz = z - cz
        d = dx * dx + dy * dy + dz * dz
        dist_acc = jnp.minimum(dist_acc, d)
        mx = jnp.max(dist_acc, axis=1, keepdims=True)
        far = jnp.min(jnp.where(dist_acc == mx, col, N), axis=1, keepdims=True)
        return dist_acc, far, cxa, cya, cza

    far0 = jnp.zeros((B, 1), jnp.int32)
    dist0 = jnp.full((B, N), 1e10, jnp.float32)
    zg = jnp.zeros((B, G), jnp.float32)
    _, _, cxa, cya, cza = jax.lax.fori_loop(0, G, step, (dist0, far0, zg, zg, zg))
    cent_ref[0] = cxa
    cent_ref[1] = cya
    cent_ref[2] = cza


def _fps(xt, interpret=False):
    return pl.pallas_call(
        _fps_body,
        out_shape=jax.ShapeDtypeStruct((3, B, G), jnp.float32),
        interpret=interpret,
    )(xt)


def kernel(pts):
    xt = jnp.transpose(pts, (2, 0, 1))
    cent = _fps(xt)
    center = jnp.transpose(cent, (1, 2, 0))

    sq = (
        jnp.sum(center**2, axis=-1, keepdims=True)
        + jnp.sum(pts**2, axis=-1)[:, None, :]
        - 2.0 * jnp.einsum("bgc,bnc->bgn", center, pts)
    )
    _, idx = jax.lax.top_k(-sq, K)
    idx_base = jnp.arange(B, dtype=idx.dtype)[:, None, None] * N
    idx_flat = (idx + idx_base).reshape(-1)
    neighborhood = pts.reshape(B * N, -1)[idx_flat, :]
    neighborhood = neighborhood.reshape(B, G, K, -1)
    neighborhood = neighborhood - center[:, :, None, :]
    return neighborhood, center


# speedup vs baseline: 10.1861x; 5.2640x over previous
import functools

import jax
import jax.numpy as jnp
from jax import lax
from jax.experimental import pallas as pl
from jax.experimental.pallas import tpu as pltpu
from jax.experimental.pallas import tpu_sc as plsc

B, N, G, K = 8, 8192, 256, 32
L = 16
NV = N // L
ROWS_PER_TILE = (B * G) // 32
GCHUNK = G // (32 // B)


def _fps_body(xt_ref, cent_ref):
    x = xt_ref[0]
    y = xt_ref[1]
    z = xt_ref[2]
    col = jax.lax.broadcasted_iota(jnp.int32, (B, N), 1)
    col_g = jax.lax.broadcasted_iota(jnp.int32, (B, G), 1)

    def step(i, carry):
        dist_acc, far, cxa, cya, cza = carry
        m = col == far
        cx = jnp.sum(jnp.where(m, x, 0.0), axis=1, keepdims=True)
        cy = jnp.sum(jnp.where(m, y, 0.0), axis=1, keepdims=True)
        cz = jnp.sum(jnp.where(m, z, 0.0), axis=1, keepdims=True)
        sel = col_g == i
        cxa = jnp.where(sel, cx, cxa)
        cya = jnp.where(sel, cy, cya)
        cza = jnp.where(sel, cz, cza)
        dx = x - cx
        dy = y - cy
        dz = z - cz
        d = dx * dx + dy * dy + dz * dz
        dist_acc = jnp.minimum(dist_acc, d)
        mx = jnp.max(dist_acc, axis=1, keepdims=True)
        far = jnp.min(jnp.where(dist_acc == mx, col, N), axis=1, keepdims=True)
        return dist_acc, far, cxa, cya, cza

    far0 = jnp.zeros((B, 1), jnp.int32)
    dist0 = jnp.full((B, N), 1e10, jnp.float32)
    zg = jnp.zeros((B, G), jnp.float32)
    _, _, cxa, cya, cza = jax.lax.fori_loop(0, G, step, (dist0, far0, zg, zg, zg))
    cent_ref[0] = cxa
    cent_ref[1] = cya
    cent_ref[2] = cza


def _fps(xt, interpret=False):
    return pl.pallas_call(
        _fps_body,
        out_shape=jax.ShapeDtypeStruct((3, B, G), jnp.float32),
        interpret=interpret,
    )(xt)


_INF = float("inf")
_BIGIDX = 1 << 30


def _kmerge(ka, va, kb, vb):
    take_a = (ka < kb) | ((ka == kb) & (va < vb))
    kmin = jnp.where(take_a, ka, kb)
    vmin = jnp.where(take_a, va, vb)
    kmax = jnp.where(take_a, kb, ka)
    vmax = jnp.where(take_a, vb, va)
    return kmin, vmin, kmax, vmax


def _sort16(k, v):
    return plsc.sort_key_val(k, v)


def _merge_sorted16(ka, va, kb, vb):
    kbr = lax.rev(kb, (0,))
    vbr = lax.rev(vb, (0,))
    kmin, vmin, kmax, vmax = _kmerge(ka, va, kbr, vbr)
    klo, vlo = _sort16(kmin, vmin)
    khi, vhi = _sort16(kmax, vmax)
    return klo, vlo, khi, vhi


def _bf16r(x):
    u = plsc.bitcast(x, jnp.uint32)
    r = (u + 0x7FFF + ((u >> 16) & 1)) & jnp.uint32(0xFFFF0000)
    return plsc.bitcast(r, jnp.float32)


def _knn_body(xt_hbm, cent_hbm, out_hbm, px, py, pz, pxb, pyb, pzb, pp,
              cxr, cyr, czr, dbuf, cdbuf, cibuf, obuf, sem):
    nc = 2
    wid = lax.axis_index("s") * nc + lax.axis_index("c")
    b = wid // 4
    g0 = (wid % 4) * GCHUNK

    pltpu.sync_copy(xt_hbm.at[0, b], px)
    pltpu.sync_copy(xt_hbm.at[1, b], py)
    pltpu.sync_copy(xt_hbm.at[2, b], pz)
    pltpu.sync_copy(cent_hbm.at[0, b, pl.ds(g0, GCHUNK)], cxr)
    pltpu.sync_copy(cent_hbm.at[1, b, pl.ds(g0, GCHUNK)], cyr)
    pltpu.sync_copy(cent_hbm.at[2, b, pl.ds(g0, GCHUNK)], czr)

    def pp_step(j, _):
        s = pl.ds(j * L, L)
        x = px[s]
        y = py[s]
        z = pz[s]
        pp[s] = (x * x + y * y) + z * z
        pxb[s] = _bf16r(x)
        pyb[s] = _bf16r(y)
        pzb[s] = _bf16r(z)
        return 0

    lax.fori_loop(0, NV, pp_step, 0, unroll=8)

    lane = lax.iota(jnp.int32, L)

    def row_body(r, _):
        vblk = pl.ds((r // L) * L, L)
        rl = r % L
        bidx = jnp.full((L,), rl, jnp.int32)
        cxb = lax.gather(
            cxr[vblk], bidx[:, None],
            lax.GatherDimensionNumbers((), (0,), (0,)), (1,),
            mode=lax.GatherScatterMode.PROMISE_IN_BOUNDS)
        cyb = lax.gather(
            cyr[vblk], bidx[:, None],
            lax.GatherDimensionNumbers((), (0,), (0,)), (1,),
            mode=lax.GatherScatterMode.PROMISE_IN_BOUNDS)
        czb = lax.gather(
            czr[vblk], bidx[:, None],
            lax.GatherDimensionNumbers((), (0,), (0,)), (1,),
            mode=lax.GatherScatterMode.PROMISE_IN_BOUNDS)
        ccb = (cxb * cxb + cyb * cyb) + czb * czb
        cxbr = _bf16r(cxb)
        cybr = _bf16r(cyb)
        czbr = _bf16r(czb)

        def p1_step(j, carry):
            m1, m2 = carry
            s = pl.ds(j * L, L)
            x = pxb[s]
            y = pyb[s]
            z = pzb[s]
            dot = cxbr * x + cybr * y + czbr * z
            d = (ccb + pp[s]) - (dot + dot)
            dbuf[s] = d
            m2 = jnp.minimum(m2, jnp.maximum(m1, d))
            m1 = jnp.minimum(m1, d)
            return m1, m2

        m1, m2 = lax.fori_loop(
            0, NV, p1_step,
            (jnp.full((L,), _INF), jnp.full((L,), _INF)), unroll=8)
        t = jnp.max(m2)

        def p2_step(j, off):
            s = pl.ds(j * L, L)
            d = dbuf[s]
            msk = d <= t
            mi = msk.astype(jnp.int32)
            pos = off + plsc.cumsum(mi) - 1
            plsc.store_scatter(cdbuf, (pos,), d, mask=msk)
            plsc.store_scatter(cibuf, (pos,), lane + (j * L), mask=msk)
            return off + plsc.all_reduce_population_count(msk)

        off = lax.fori_loop(0, NV, p2_step, jnp.zeros((L,), jnp.int32),
                            unroll=4)
        m = jnp.max(off)
        nv = (m + (L - 1)) // L

        klo, vlo = _sort16(cdbuf[pl.ds(0, L)], cibuf[pl.ds(0, L)])
        k1, v1 = _sort16(cdbuf[pl.ds(L, L)], cibuf[pl.ds(L, L)])
        klo, vlo, khi, vhi = _merge_sorted16(klo, vlo, k1, v1)

        def p3_step(kblk, carry):
            klo, vlo, khi, vhi = carry
            base = kblk * L
            kc = cdbuf[pl.ds(base, L)]
            vc = cibuf[pl.ds(base, L)]
            valid = (lane + base) < m
            kc = jnp.where(valid, kc, _INF)
            vc = jnp.where(valid, vc, _BIGIDX)
            kc, vc = _sort16(kc, vc)
            kcr = lax.rev(kc, (0,))
            vcr = lax.rev(vc, (0,))
            kh, vh, _, _ = _kmerge(khi, vhi, kcr, vcr)
            kh, vh = _sort16(kh, vh)
            return _merge_sorted16(klo, vlo, kh, vh)

        klo, vlo, khi, vhi = lax.fori_loop(2, nv, p3_step,
                                           (klo, vlo, khi, vhi))

        gx_lo = plsc.load_gather(px, (vlo,))
        gy_lo = plsc.load_gather(py, (vlo,))
        gz_lo = plsc.load_gather(pz, (vlo,))
        gx_hi = plsc.load_gather(px, (vhi,))
        gy_hi = plsc.load_gather(py, (vhi,))
        gz_hi = plsc.load_gather(pz, (vhi,))
        obuf[0, r, pl.ds(0, L)] = gx_lo - cxb
        obuf[0, r, pl.ds(L, L)] = gx_hi - cxb
        obuf[1, r, pl.ds(0, L)] = gy_lo - cyb
        obuf[1, r, pl.ds(L, L)] = gy_hi - cyb
        obuf[2, r, pl.ds(0, L)] = gz_lo - czb
        obuf[2, r, pl.ds(L, L)] = gz_hi - czb
        return 0

    lax.fori_loop(0, ROWS_PER_TILE, row_body, 0)

    pltpu.sync_copy(obuf.at[0], out_hbm.at[0, pl.ds(wid * ROWS_PER_TILE, ROWS_PER_TILE)])
    pltpu.sync_copy(obuf.at[1], out_hbm.at[1, pl.ds(wid * ROWS_PER_TILE, ROWS_PER_TILE)])
    pltpu.sync_copy(obuf.at[2], out_hbm.at[2, pl.ds(wid * ROWS_PER_TILE, ROWS_PER_TILE)])


def _knn_sc(xt, cent):
    mesh = plsc.VectorSubcoreMesh(core_axis_name="c", subcore_axis_name="s")
    f = pl.kernel(
        _knn_body,
        out_type=jax.ShapeDtypeStruct((3, B * G, K), jnp.float32),
        mesh=mesh,
        compiler_params=pltpu.CompilerParams(needs_layout_passes=False),
        scratch_types=[
            pltpu.VMEM((N,), jnp.float32),
            pltpu.VMEM((N,), jnp.float32),
            pltpu.VMEM((N,), jnp.float32),
            pltpu.VMEM((N,), jnp.float32),
            pltpu.VMEM((N,), jnp.float32),
            pltpu.VMEM((N,), jnp.float32),
            pltpu.VMEM((N,), jnp.float32),
            pltpu.VMEM((GCHUNK,), jnp.float32),
            pltpu.VMEM((GCHUNK,), jnp.float32),
            pltpu.VMEM((GCHUNK,), jnp.float32),
            pltpu.VMEM((N,), jnp.float32),
            pltpu.VMEM((N,), jnp.float32),
            pltpu.VMEM((N,), jnp.int32),
            pltpu.VMEM((3, ROWS_PER_TILE, K), jnp.float32),
            pltpu.SemaphoreType.DMA,
        ],
    )
    return f(xt, cent)


def kernel(pts):
    xt = jnp.transpose(pts, (2, 0, 1))
    cent = _fps(xt)
    center = jnp.transpose(cent, (1, 2, 0))
    nbr = _knn_sc(xt, cent)
    neighborhood = jnp.transpose(nbr.reshape(3, B, G, K), (1, 2, 3, 0))
    return neighborhood, center


# speedup vs baseline: 11.5203x; 1.1310x over previous
import functools

import jax
import jax.numpy as jnp
from jax import lax
from jax.experimental import pallas as pl
from jax.experimental.pallas import tpu as pltpu
from jax.experimental.pallas import tpu_sc as plsc

B, N, G, K = 8, 8192, 256, 32
L = 16
NV = N // L
ROWS_PER_TILE = (B * G) // 32
GCHUNK = G // (32 // B)


def _fps_body(xt_ref, cent_ref):
    x = xt_ref[0]
    y = xt_ref[1]
    z = xt_ref[2]
    col = jax.lax.broadcasted_iota(jnp.int32, (B, N), 1)
    col_g = jax.lax.broadcasted_iota(jnp.int32, (B, G), 1)

    def step(i, carry):
        dist_acc, far, cxa, cya, cza = carry
        m = col == far
        cx = jnp.sum(jnp.where(m, x, 0.0), axis=1, keepdims=True)
        cy = jnp.sum(jnp.where(m, y, 0.0), axis=1, keepdims=True)
        cz = jnp.sum(jnp.where(m, z, 0.0), axis=1, keepdims=True)
        sel = col_g == i
        cxa = jnp.where(sel, cx, cxa)
        cya = jnp.where(sel, cy, cya)
        cza = jnp.where(sel, cz, cza)
        dx = x - cx
        dy = y - cy
        dz = z - cz
        d = dx * dx + dy * dy + dz * dz
        dist_acc = jnp.minimum(dist_acc, d)
        mx = jnp.max(dist_acc, axis=1, keepdims=True)
        far = jnp.min(jnp.where(dist_acc == mx, col, N), axis=1, keepdims=True)
        return dist_acc, far, cxa, cya, cza

    far0 = jnp.zeros((B, 1), jnp.int32)
    dist0 = jnp.full((B, N), 1e10, jnp.float32)
    zg = jnp.zeros((B, G), jnp.float32)
    _, _, cxa, cya, cza = jax.lax.fori_loop(0, G, step, (dist0, far0, zg, zg, zg))
    cent_ref[0] = cxa
    cent_ref[1] = cya
    cent_ref[2] = cza


def _fps(xt, interpret=False):
    return pl.pallas_call(
        _fps_body,
        out_shape=jax.ShapeDtypeStruct((3, B, G), jnp.float32),
        interpret=interpret,
    )(xt)


_INF = float("inf")
_BIGIDX = 1 << 30


def _kmerge(ka, va, kb, vb):
    take_a = (ka < kb) | ((ka == kb) & (va < vb))
    kmin = jnp.where(take_a, ka, kb)
    vmin = jnp.where(take_a, va, vb)
    kmax = jnp.where(take_a, kb, ka)
    vmax = jnp.where(take_a, vb, va)
    return kmin, vmin, kmax, vmax


def _sort16(k, v):
    return plsc.sort_key_val(k, v)


def _merge_sorted16(ka, va, kb, vb):
    kbr = lax.rev(kb, (0,))
    vbr = lax.rev(vb, (0,))
    kmin, vmin, kmax, vmax = _kmerge(ka, va, kbr, vbr)
    klo, vlo = _sort16(kmin, vmin)
    khi, vhi = _sort16(kmax, vmax)
    return klo, vlo, khi, vhi


def _bf16r(x):
    u = plsc.bitcast(x, jnp.uint32)
    r = (u + 0x7FFF + ((u >> 16) & 1)) & jnp.uint32(0xFFFF0000)
    return plsc.bitcast(r, jnp.float32)


def _knn_body(xt_hbm, cent_hbm, out_hbm, px, py, pz, pxb, pyb, pzb, pp,
              cxr, cyr, czr, dbuf, cdbuf, cibuf, obuf, sem):
    nc = 2
    wid = lax.axis_index("s") * nc + lax.axis_index("c")
    b = wid // 4
    g0 = (wid % 4) * GCHUNK

    pltpu.sync_copy(xt_hbm.at[0, b], px)
    pltpu.sync_copy(xt_hbm.at[1, b], py)
    pltpu.sync_copy(xt_hbm.at[2, b], pz)
    pltpu.sync_copy(cent_hbm.at[0, b, pl.ds(g0, GCHUNK)], cxr)
    pltpu.sync_copy(cent_hbm.at[1, b, pl.ds(g0, GCHUNK)], cyr)
    pltpu.sync_copy(cent_hbm.at[2, b, pl.ds(g0, GCHUNK)], czr)

    def pp_step(j, _):
        s = pl.ds(j * L, L)
        x = px[s]
        y = py[s]
        z = pz[s]
        pp[s] = (x * x + y * y) + z * z
        pxb[s] = _bf16r(x)
        pyb[s] = _bf16r(y)
        pzb[s] = _bf16r(z)
        return 0

    lax.fori_loop(0, NV, pp_step, 0, unroll=8)

    lane = lax.iota(jnp.int32, L)

    def row_body(r, _):
        vblk = pl.ds((r // L) * L, L)
        rl = r % L
        bidx = jnp.full((L,), rl, jnp.int32)
        cxb = lax.gather(
            cxr[vblk], bidx[:, None],
            lax.GatherDimensionNumbers((), (0,), (0,)), (1,),
            mode=lax.GatherScatterMode.PROMISE_IN_BOUNDS)
        cyb = lax.gather(
            cyr[vblk], bidx[:, None],
            lax.GatherDimensionNumbers((), (0,), (0,)), (1,),
            mode=lax.GatherScatterMode.PROMISE_IN_BOUNDS)
        czb = lax.gather(
            czr[vblk], bidx[:, None],
            lax.GatherDimensionNumbers((), (0,), (0,)), (1,),
            mode=lax.GatherScatterMode.PROMISE_IN_BOUNDS)
        ccb = (cxb * cxb + cyb * cyb) + czb * czb
        cxbr = _bf16r(cxb)
        cybr = _bf16r(cyb)
        czbr = _bf16r(czb)

        def p1_step(j, carry):
            m1, m2 = carry
            s = pl.ds(j * L, L)
            x = pxb[s]
            y = pyb[s]
            z = pzb[s]
            dot = cxbr * x + cybr * y + czbr * z
            d = (ccb + pp[s]) - (dot + dot)
            dbuf[s] = d
            m2 = jnp.minimum(m2, jnp.maximum(m1, d))
            m1 = jnp.minimum(m1, d)
            return m1, m2

        m1, m2 = lax.fori_loop(
            0, NV, p1_step,
            (jnp.full((L,), _INF), jnp.full((L,), _INF)), unroll=8)
        t = jnp.max(m2)

        def p2_step(j, off):
            s = pl.ds(j * L, L)
            d = dbuf[s]
            msk = d <= t
            plsc.store_compressed(cdbuf.at[pl.ds(off, L)], d, mask=msk)
            plsc.store_compressed(cibuf.at[pl.ds(off, L)], lane + (j * L), mask=msk)
            return off + plsc.all_reduce_population_count(msk)[0]

        m = lax.fori_loop(0, NV, p2_step, jnp.int32(0), unroll=8)
        nv = (m + (L - 1)) // L

        klo, vlo = _sort16(cdbuf[pl.ds(0, L)], cibuf[pl.ds(0, L)])
        k1, v1 = _sort16(cdbuf[pl.ds(L, L)], cibuf[pl.ds(L, L)])
        klo, vlo, khi, vhi = _merge_sorted16(klo, vlo, k1, v1)

        def p3_step(kblk, carry):
            klo, vlo, khi, vhi = carry
            base = kblk * L
            kc = cdbuf[pl.ds(base, L)]
            vc = cibuf[pl.ds(base, L)]
            valid = (lane + base) < m
            kc = jnp.where(valid, kc, _INF)
            vc = jnp.where(valid, vc, _BIGIDX)
            kc, vc = _sort16(kc, vc)
            kcr = lax.rev(kc, (0,))
            vcr = lax.rev(vc, (0,))
            kh, vh, _, _ = _kmerge(khi, vhi, kcr, vcr)
            kh, vh = _sort16(kh, vh)
            return _merge_sorted16(klo, vlo, kh, vh)

        klo, vlo, khi, vhi = lax.fori_loop(2, nv, p3_step,
                                           (klo, vlo, khi, vhi))

        gx_lo = plsc.load_gather(px, (vlo,))
        gy_lo = plsc.load_gather(py, (vlo,))
        gz_lo = plsc.load_gather(pz, (vlo,))
        gx_hi = plsc.load_gather(px, (vhi,))
        gy_hi = plsc.load_gather(py, (vhi,))
        gz_hi = plsc.load_gather(pz, (vhi,))
        obuf[0, r, pl.ds(0, L)] = gx_lo - cxb
        obuf[0, r, pl.ds(L, L)] = gx_hi - cxb
        obuf[1, r, pl.ds(0, L)] = gy_lo - cyb
        obuf[1, r, pl.ds(L, L)] = gy_hi - cyb
        obuf[2, r, pl.ds(0, L)] = gz_lo - czb
        obuf[2, r, pl.ds(L, L)] = gz_hi - czb
        return 0

    lax.fori_loop(0, ROWS_PER_TILE, row_body, 0)

    pltpu.sync_copy(obuf.at[0], out_hbm.at[0, pl.ds(wid * ROWS_PER_TILE, ROWS_PER_TILE)])
    pltpu.sync_copy(obuf.at[1], out_hbm.at[1, pl.ds(wid * ROWS_PER_TILE, ROWS_PER_TILE)])
    pltpu.sync_copy(obuf.at[2], out_hbm.at[2, pl.ds(wid * ROWS_PER_TILE, ROWS_PER_TILE)])


def _knn_sc(xt, cent):
    mesh = plsc.VectorSubcoreMesh(core_axis_name="c", subcore_axis_name="s")
    f = pl.kernel(
        _knn_body,
        out_type=jax.ShapeDtypeStruct((3, B * G, K), jnp.float32),
        mesh=mesh,
        compiler_params=pltpu.CompilerParams(needs_layout_passes=False),
        scratch_types=[
            pltpu.VMEM((N,), jnp.float32),
            pltpu.VMEM((N,), jnp.float32),
            pltpu.VMEM((N,), jnp.float32),
            pltpu.VMEM((N,), jnp.float32),
            pltpu.VMEM((N,), jnp.float32),
            pltpu.VMEM((N,), jnp.float32),
            pltpu.VMEM((N,), jnp.float32),
            pltpu.VMEM((GCHUNK,), jnp.float32),
            pltpu.VMEM((GCHUNK,), jnp.float32),
            pltpu.VMEM((GCHUNK,), jnp.float32),
            pltpu.VMEM((N,), jnp.float32),
            pltpu.VMEM((N + L,), jnp.float32),
            pltpu.VMEM((N + L,), jnp.int32),
            pltpu.VMEM((3, ROWS_PER_TILE, K), jnp.float32),
            pltpu.SemaphoreType.DMA,
        ],
    )
    return f(xt, cent)


def kernel(pts):
    xt = jnp.transpose(pts, (2, 0, 1))
    cent = _fps(xt)
    center = jnp.transpose(cent, (1, 2, 0))
    nbr = _knn_sc(xt, cent)
    neighborhood = jnp.transpose(nbr.reshape(3, B, G, K), (1, 2, 3, 0))
    return neighborhood, center


# speedup vs baseline: 18.5763x; 1.6125x over previous
import functools

import jax
import jax.numpy as jnp
from jax import lax
from jax.experimental import pallas as pl
from jax.experimental.pallas import tpu as pltpu
from jax.experimental.pallas import tpu_sc as plsc

B, N, G, K = 8, 8192, 256, 32
L = 16
NV = N // L
ROWS_PER_TILE = (B * G) // 32
GCHUNK = G // (32 // B)


def _fps_body(xt_ref, cent_ref):
    x = xt_ref[0]
    y = xt_ref[1]
    z = xt_ref[2]
    col = jax.lax.broadcasted_iota(jnp.int32, (B, N), 1)
    col_g = jax.lax.broadcasted_iota(jnp.int32, (B, G), 1)

    def step(i, carry):
        dist_acc, far, cxa, cya, cza = carry
        m = col == far
        cx = jnp.sum(jnp.where(m, x, 0.0), axis=1, keepdims=True)
        cy = jnp.sum(jnp.where(m, y, 0.0), axis=1, keepdims=True)
        cz = jnp.sum(jnp.where(m, z, 0.0), axis=1, keepdims=True)
        sel = col_g == i
        cxa = jnp.where(sel, cx, cxa)
        cya = jnp.where(sel, cy, cya)
        cza = jnp.where(sel, cz, cza)
        dx = x - cx
        dy = y - cy
        dz = z - cz
        d = dx * dx + dy * dy + dz * dz
        dist_acc = jnp.minimum(dist_acc, d)
        mx = jnp.max(dist_acc, axis=1, keepdims=True)
        far = jnp.min(jnp.where(dist_acc == mx, col, N), axis=1, keepdims=True)
        return dist_acc, far, cxa, cya, cza

    far0 = jnp.zeros((B, 1), jnp.int32)
    dist0 = jnp.full((B, N), 1e10, jnp.float32)
    zg = jnp.zeros((B, G), jnp.float32)
    _, _, cxa, cya, cza = jax.lax.fori_loop(0, G, step, (dist0, far0, zg, zg, zg))
    cent_ref[0] = cxa
    cent_ref[1] = cya
    cent_ref[2] = cza


def _fps(xt, interpret=False):
    return pl.pallas_call(
        _fps_body,
        out_shape=jax.ShapeDtypeStruct((3, B, G), jnp.float32),
        interpret=interpret,
    )(xt)


_INF = float("inf")
_BIGIDX = 1 << 30


def _kmerge(ka, va, kb, vb):
    take_a = (ka < kb) | ((ka == kb) & (va < vb))
    kmin = jnp.where(take_a, ka, kb)
    vmin = jnp.where(take_a, va, vb)
    kmax = jnp.where(take_a, kb, ka)
    vmax = jnp.where(take_a, vb, va)
    return kmin, vmin, kmax, vmax


def _sort16(k, v):
    return plsc.sort_key_val(k, v)


def _merge_sorted16(ka, va, kb, vb):
    kbr = lax.rev(kb, (0,))
    vbr = lax.rev(vb, (0,))
    kmin, vmin, kmax, vmax = _kmerge(ka, va, kbr, vbr)
    klo, vlo = _sort16(kmin, vmin)
    khi, vhi = _sort16(kmax, vmax)
    return klo, vlo, khi, vhi


def _bf16r(x):
    u = plsc.bitcast(x, jnp.uint32)
    r = (u + 0x7FFF + ((u >> 16) & 1)) & jnp.uint32(0xFFFF0000)
    return plsc.bitcast(r, jnp.float32)


def _treemax(v):
    m = v
    for sh in (8, 4, 2, 1):
        perm = lax.iota(jnp.int32, L) ^ sh
        g = lax.gather(m, perm[:, None],
                       lax.GatherDimensionNumbers((), (0,), (0,)), (1,),
                       mode=lax.GatherScatterMode.PROMISE_IN_BOUNDS)
        m = jnp.maximum(m, g)
    return m


def _knn_body(xt_hbm, cent_hbm, out_hbm, px, py, pz, pxb, pyb, pzb, pp,
              cxr, cyr, czr, cda, cia, cdb, cib, obuf, sem):
    nc = 2
    wid = lax.axis_index("s") * nc + lax.axis_index("c")
    b = wid // 4
    g0 = (wid % 4) * GCHUNK

    pltpu.sync_copy(xt_hbm.at[0, b], px)
    pltpu.sync_copy(xt_hbm.at[1, b], py)
    pltpu.sync_copy(xt_hbm.at[2, b], pz)
    pltpu.sync_copy(cent_hbm.at[0, b, pl.ds(g0, GCHUNK)], cxr)
    pltpu.sync_copy(cent_hbm.at[1, b, pl.ds(g0, GCHUNK)], cyr)
    pltpu.sync_copy(cent_hbm.at[2, b, pl.ds(g0, GCHUNK)], czr)

    def pp_step(j, _):
        s = pl.ds(j * L, L)
        x = px[s]
        y = py[s]
        z = pz[s]
        pp[s] = (x * x + y * y) + z * z
        pxb[s] = _bf16r(x)
        pyb[s] = _bf16r(y)
        pzb[s] = _bf16r(z)
        return 0

    lax.fori_loop(0, NV, pp_step, 0, unroll=8)

    lane = lax.iota(jnp.int32, L)
    inf16 = jnp.full((L,), _INF)

    def crow(r):
        vblk = pl.ds((r // L) * L, L)
        bidx = jnp.full((L,), r % L, jnp.int32)

        def bc(ref):
            return lax.gather(
                ref[vblk], bidx[:, None],
                lax.GatherDimensionNumbers((), (0,), (0,)), (1,),
                mode=lax.GatherScatterMode.PROMISE_IN_BOUNDS)

        cxv = bc(cxr)
        cyv = bc(cyr)
        czv = bc(czr)
        ccv = (cxv * cxv + cyv * cyv) + czv * czv
        return cxv, cyv, czv, ccv, _bf16r(cxv), _bf16r(cyv), _bf16r(czv)

    UN = 8

    def finish(cxv, cyv, czv, ccv, cxq, cyq, czq, m2, moff, cd_, ci_, rout):
        tf = _treemax(m2)

        def rc_step(kb, off2):
            base = kb * L
            idx = ci_[pl.ds(base, L)]
            valid = (lane + base) < moff
            idxs = jnp.where(valid, idx, 0)
            gx = plsc.load_gather(pxb, (idxs,))
            gy = plsc.load_gather(pyb, (idxs,))
            gz = plsc.load_gather(pzb, (idxs,))
            gq = plsc.load_gather(pp, (idxs,))
            dot = cxq * gx + cyq * gy + czq * gz
            d = (ccv + gq) - (dot + dot)
            msk = (d <= tf) & valid
            plsc.store_compressed(cd_.at[pl.ds(off2, L)], d, mask=msk)
            plsc.store_compressed(ci_.at[pl.ds(off2, L)], idx, mask=msk)
            return off2 + plsc.all_reduce_population_count(msk)[0]

        nva = (moff + (L - 1)) // L
        m = lax.fori_loop(0, nva, rc_step, jnp.int32(0))
        nv = (m + (L - 1)) // L

        klo, vlo = _sort16(cd_[pl.ds(0, L)], ci_[pl.ds(0, L)])
        k1, v1 = _sort16(cd_[pl.ds(L, L)], ci_[pl.ds(L, L)])
        klo, vlo, khi, vhi = _merge_sorted16(klo, vlo, k1, v1)

        def p3_step(kblk, carry):
            klo, vlo, khi, vhi = carry
            base = kblk * L
            kc = cd_[pl.ds(base, L)]
            vc = ci_[pl.ds(base, L)]
            valid = (lane + base) < m
            kc = jnp.where(valid, kc, _INF)
            vc = jnp.where(valid, vc, _BIGIDX)
            kc, vc = _sort16(kc, vc)
            kcr = lax.rev(kc, (0,))
            vcr = lax.rev(vc, (0,))
            kh, vh, _, _ = _kmerge(khi, vhi, kcr, vcr)
            kh, vh = _sort16(kh, vh)
            return _merge_sorted16(klo, vlo, kh, vh)

        klo, vlo, khi, vhi = lax.fori_loop(2, nv, p3_step,
                                           (klo, vlo, khi, vhi))

        gx_lo = plsc.load_gather(px, (vlo,))
        gy_lo = plsc.load_gather(py, (vlo,))
        gz_lo = plsc.load_gather(pz, (vlo,))
        gx_hi = plsc.load_gather(px, (vhi,))
        gy_hi = plsc.load_gather(py, (vhi,))
        gz_hi = plsc.load_gather(pz, (vhi,))
        obuf[0, rout, pl.ds(0, L)] = gx_lo - cxv
        obuf[0, rout, pl.ds(L, L)] = gx_hi - cxv
        obuf[1, rout, pl.ds(0, L)] = gy_lo - cyv
        obuf[1, rout, pl.ds(L, L)] = gy_hi - cyv
        obuf[2, rout, pl.ds(0, L)] = gz_lo - czv
        obuf[2, rout, pl.ds(L, L)] = gz_hi - czv

    def pair_body(h, _):
        ra = 2 * h
        rb = ra + 1
        cxa, cya, cza, cca, cxqa, cyqa, czqa = crow(ra)
        cxb2, cyb2, czb2, ccb2, cxqb, cyqb, czqb = crow(rb)

        def grp_body(g, carry):
            m1a, m2a, m1b, m2b, ta, tb, offa, offb = carry
            for u in range(UN):
                j = g * UN + u
                s = pl.ds(j * L, L)
                x = pxb[s]
                y = pyb[s]
                z = pzb[s]
                q = pp[s]
                dota = cxqa * x + cyqa * y + czqa * z
                da = (cca + q) - (dota + dota)
                dotb = cxqb * x + cyqb * y + czqb * z
                db = (ccb2 + q) - (dotb + dotb)
                m2a = jnp.minimum(m2a, jnp.maximum(m1a, da))
                m1a = jnp.minimum(m1a, da)
                m2b = jnp.minimum(m2b, jnp.maximum(m1b, db))
                m1b = jnp.minimum(m1b, db)
                idxv = lane + (j * L)
                mka = da <= ta
                plsc.store_compressed(cia.at[pl.ds(offa, L)], idxv, mask=mka)
                offa = offa + plsc.all_reduce_population_count(mka)[0]
                mkb = db <= tb
                plsc.store_compressed(cib.at[pl.ds(offb, L)], idxv, mask=mkb)
                offb = offb + plsc.all_reduce_population_count(mkb)[0]
            ta = _treemax(m2a)
            tb = _treemax(m2b)
            return (m1a, m2a, m1b, m2b, ta, tb, offa, offb)

        carry0 = (inf16, inf16, inf16, inf16, inf16, inf16,
                  jnp.int32(0), jnp.int32(0))
        _, m2a, _, m2b, _, _, offa, offb = lax.fori_loop(
            0, NV // UN, grp_body, carry0)

        finish(cxa, cya, cza, cca, cxqa, cyqa, czqa, m2a, offa, cda, cia, ra)
        finish(cxb2, cyb2, czb2, ccb2, cxqb, cyqb, czqb, m2b, offb, cdb, cib, rb)
        return 0

    lax.fori_loop(0, ROWS_PER_TILE // 2, pair_body, 0)

    pltpu.sync_copy(obuf.at[0], out_hbm.at[0, pl.ds(wid * ROWS_PER_TILE, ROWS_PER_TILE)])
    pltpu.sync_copy(obuf.at[1], out_hbm.at[1, pl.ds(wid * ROWS_PER_TILE, ROWS_PER_TILE)])
    pltpu.sync_copy(obuf.at[2], out_hbm.at[2, pl.ds(wid * ROWS_PER_TILE, ROWS_PER_TILE)])


def _knn_sc(xt, cent):
    mesh = plsc.VectorSubcoreMesh(core_axis_name="c", subcore_axis_name="s")
    f = pl.kernel(
        _knn_body,
        out_type=jax.ShapeDtypeStruct((3, B * G, K), jnp.float32),
        mesh=mesh,
        compiler_params=pltpu.CompilerParams(needs_layout_passes=False),
        scratch_types=[
            pltpu.VMEM((N,), jnp.float32),
            pltpu.VMEM((N,), jnp.float32),
            pltpu.VMEM((N,), jnp.float32),
            pltpu.VMEM((N,), jnp.float32),
            pltpu.VMEM((N,), jnp.float32),
            pltpu.VMEM((N,), jnp.float32),
            pltpu.VMEM((N,), jnp.float32),
            pltpu.VMEM((GCHUNK,), jnp.float32),
            pltpu.VMEM((GCHUNK,), jnp.float32),
            pltpu.VMEM((GCHUNK,), jnp.float32),
            pltpu.VMEM((N + L,), jnp.float32),
            pltpu.VMEM((N + L,), jnp.int32),
            pltpu.VMEM((N + L,), jnp.float32),
            pltpu.VMEM((N + L,), jnp.int32),
            pltpu.VMEM((3, ROWS_PER_TILE, K), jnp.float32),
            pltpu.SemaphoreType.DMA,
        ],
    )
    return f(xt, cent)


def kernel(pts):
    xt = jnp.transpose(pts, (2, 0, 1))
    cent = _fps(xt)
    center = jnp.transpose(cent, (1, 2, 0))
    nbr = _knn_sc(xt, cent)
    neighborhood = jnp.transpose(nbr.reshape(3, B, G, K), (1, 2, 3, 0))
    return neighborhood, center
